# Initial kernel scaffold; baseline (speedup 1.0000x reference)
import jax
import jax.numpy as jnp
from jax.experimental import pallas as pl


def kernel(x, features, lags, Wk, bk, Wg, bg, Wgg, bgg, overlap_win):
    raise NotImplementedError("write your pallas kernel here")


# speedup vs baseline: 4.1252x; 4.1252x over previous
import functools
import jax
import jax.numpy as jnp
from jax.experimental import pallas as pl
from jax.experimental.pallas import tpu as pltpu

KS = 15
FD = 96
FS = 160
OV = 40
ML = 256
PADL = 7
LGL = 10 * 0.11512925464970229
FGA = 6 * 0.11512925464970229
NF = 100
L_OUT = FS + OV
XX_W = 224
ME = 22
XP_LEN = 16512


def _tc_kernel(lags_pref, xp_ref, feats_ref, xs_ref, wall_ref,
               ball_ref, win1_ref, win2_ref, out_ref, xx_ref):
    b = pl.program_id(0)

    def body(i, _):
        lag = lags_pref[b * NF + i]
        start = i * FS + ML - lag
        r = start & 127
        astart = pl.multiple_of(start - r, 128)
        w = xp_ref[0, pl.ds(0, 1), pl.ds(astart, 384)]
        rolled = pltpu.roll(w, 384 - r, 1)
        xx_ref[pl.ds(i, 1), :] = rolled[:, :XX_W]
        return 0

    jax.lax.fori_loop(0, NF, body, 0)

    feats = feats_ref[0]
    p = jnp.dot(feats, wall_ref[...], preferred_element_type=jnp.float32)
    p = p + ball_ref[...]
    ck = p[:, :KS]
    nrm = jnp.sqrt(jnp.sum(ck * ck, axis=-1, keepdims=True))
    ck = ck / (1e-6 + nrm)
    cg = jnp.exp(-jax.nn.relu(p[:, KS:KS + 1]) + LGL)
    gg = jnp.exp(FGA * jnp.tanh(p[:, KS + 1:KS + 2]))

    xx = xx_ref[...]
    nc = jnp.zeros((NF, L_OUT), dtype=jnp.float32)
    for m in range(KS):
        nc = nc + ck[:, m:m + 1] * xx[:, m:m + L_OUT]

    nc = gg * (nc * cg + xs_ref[0])

    prev_tail = jnp.concatenate(
        [jnp.zeros((1, OV), jnp.float32), nc[:NF - 1, FS:FS + OV]], axis=0)
    head = nc[:, :OV] * win1_ref[...] + prev_tail * win2_ref[...]
    out_ref[0] = jnp.concatenate([head, nc[:, OV:FS]], axis=1)


def kernel(x, features, lags, Wk, bk, Wg, bg, Wgg, bgg, overlap_win):
    B = x.shape[0]
    lags = lags.astype(jnp.int32)

    xp = jnp.zeros((B, XP_LEN), dtype=jnp.float32)
    xp = jax.lax.dynamic_update_slice(xp, x[:, 0, :], (0, ML + PADL))
    xp = xp.reshape(B, 1, XP_LEN)

    xr = x[:, 0, :].reshape(B, NF, FS)
    nxt = jnp.concatenate(
        [xr[:, 1:, :OV], jnp.zeros((B, 1, OV), jnp.float32)], axis=1)
    xs = jnp.concatenate([xr, nxt], axis=-1)

    wall = jnp.concatenate([Wk, Wg, Wgg], axis=0).T
    ball = jnp.concatenate(
        [bk, bg, bgg]).astype(jnp.float32).reshape(1, KS + 2)
    win2 = overlap_win.reshape(1, OV)
    win1 = win2[:, ::-1]

    grid_spec = pltpu.PrefetchScalarGridSpec(
        num_scalar_prefetch=1,
        grid=(B,),
        in_specs=[
            pl.BlockSpec((1, 1, XP_LEN), lambda b, lp: (b, 0, 0)),
            pl.BlockSpec((1, NF, FD), lambda b, lp: (b, 0, 0)),
            pl.BlockSpec((1, NF, L_OUT), lambda b, lp: (b, 0, 0)),
            pl.BlockSpec((FD, KS + 2), lambda b, lp: (0, 0)),
            pl.BlockSpec((1, KS + 2), lambda b, lp: (0, 0)),
            pl.BlockSpec((1, OV), lambda b, lp: (0, 0)),
            pl.BlockSpec((1, OV), lambda b, lp: (0, 0)),
        ],
        out_specs=pl.BlockSpec((1, NF, FS), lambda b, lp: (b, 0, 0)),
        scratch_shapes=[pltpu.VMEM((NF, XX_W), jnp.float32)],
    )

    out = pl.pallas_call(
        _tc_kernel,
        grid_spec=grid_spec,
        out_shape=jax.ShapeDtypeStruct((B, NF, FS), jnp.float32),
    )(lags.reshape(B * NF), xp, features, xs, wall, ball, win1, win2)

    return out.reshape(B, 1, NF * FS)


# speedup vs baseline: 7.9416x; 1.9251x over previous
import jax
import jax.numpy as jnp
from jax.experimental import pallas as pl
from jax.experimental.pallas import tpu as pltpu
from jax.experimental.pallas import tpu_sc as plsc

KS = 15
FD = 96
FS = 160
OV = 40
ML = 256
PADL = 7
LGL = 10 * 0.11512925464970229
FGA = 6 * 0.11512925464970229
NF = 100
L_OUT = FS + OV
RW = 128
RPP = 3
XX_W = RPP * RW
XP_LEN = 16512
XP_ROWS = XP_LEN // RW
PAIRS_PAD = 112
NIDX = RPP * PAIRS_PAD
IDX_CH = 3


def _sc_gather(xp_rows, lags_hbm, xx_hbm, lag_v, idx_v, buf_v, sem):
    c = jax.lax.axis_index("c")
    s = jax.lax.axis_index("s")
    b = s * 2 + c
    pltpu.sync_copy(lags_hbm.at[b], lag_v)

    zero16 = jnp.zeros((16,), jnp.int32)
    for z in range((IDX_CH * 128 - NIDX) // 16):
        idx_v[pl.ds(NIDX + z * 16, 16)] = zero16

    base = b * XP_ROWS
    for k in range(PAIRS_PAD // 16):
        i16 = jax.lax.iota(jnp.int32, 16) + (k * 16)
        lag16 = lag_v[pl.ds(k * 16, 16)]
        start16 = i16 * FS + ML - lag16
        ar16 = jax.lax.shift_right_logical(start16, 7)
        row16 = jnp.where(i16 < NF, base + ar16, base)
        for r in range(RPP):
            idx_v[pl.ds(r * PAIRS_PAD + k * 16, 16)] = row16 + r

    copies = []
    for j in range(IDX_CH):
        cp = pltpu.make_async_copy(
            xp_rows.at[idx_v.at[pl.ds(j * 128, 128)]],
            buf_v.at[pl.ds(j * 128, 128)], sem)
        cp.start()
        copies.append(cp)
    for cp in copies:
        cp.wait()

    pltpu.sync_copy(buf_v.at[pl.ds(0, NIDX)], xx_hbm.at[b])


def _tc_kernel(xx_ref, feats_ref, lagsv_ref, xs_ref, wall_ref,
               ball_ref, win1_ref, win2_ref, out_ref):
    feats = feats_ref[0]
    p = jnp.dot(feats, wall_ref[...], preferred_element_type=jnp.float32)
    p = p + ball_ref[...]
    ck = p[:, :KS]
    nrm = jnp.sqrt(jnp.sum(ck * ck, axis=-1, keepdims=True))
    ck = ck / (1e-6 + nrm)
    cg = jnp.exp(-jax.nn.relu(p[:, KS:KS + 1]) + LGL)
    gg = jnp.exp(FGA * jnp.tanh(p[:, KS + 1:KS + 2]))

    w = jnp.concatenate(
        [xx_ref[0, r] for r in range(RPP)], axis=1)[:NF]

    lags = lagsv_ref[0, :, :]
    i_col = jax.lax.broadcasted_iota(jnp.int32, (NF, 1), 0)
    rem = (i_col * FS + ML - lags) & (RW - 1)
    for kbit in (64, 32, 16, 8, 4, 2, 1):
        wsh = jnp.concatenate(
            [w[:, kbit:], jnp.zeros((NF, kbit), jnp.float32)], axis=1)
        w = jnp.where((rem & kbit) != 0, wsh, w)

    nc = jnp.zeros((NF, L_OUT), dtype=jnp.float32)
    for m in range(KS):
        nc = nc + ck[:, m:m + 1] * w[:, m:m + L_OUT]

    nc = gg * (nc * cg + xs_ref[0])

    prev_tail = jnp.concatenate(
        [jnp.zeros((1, OV), jnp.float32), nc[:NF - 1, FS:FS + OV]], axis=0)
    head = nc[:, :OV] * win1_ref[...] + prev_tail * win2_ref[...]
    out_ref[0] = jnp.concatenate([head, nc[:, OV:FS]], axis=1)


def kernel(x, features, lags, Wk, bk, Wg, bg, Wgg, bgg, overlap_win):
    B = x.shape[0]
    lags = lags.astype(jnp.int32)

    xp = jnp.zeros((B, XP_LEN), dtype=jnp.float32)
    xp = jax.lax.dynamic_update_slice(xp, x[:, 0, :], (0, ML + PADL))
    xp_rows = xp.reshape(B * XP_ROWS, RW)

    lags_pad = jnp.pad(lags, ((0, 0), (0, PAIRS_PAD - NF)))

    mesh = plsc.VectorSubcoreMesh(core_axis_name="c", subcore_axis_name="s")
    xx = pl.kernel(
        _sc_gather,
        out_type=jax.ShapeDtypeStruct((B, NIDX, RW), jnp.float32),
        mesh=mesh,
        scratch_types=[
            pltpu.VMEM((PAIRS_PAD,), jnp.int32),
            pltpu.VMEM((IDX_CH * 128,), jnp.int32),
            pltpu.VMEM((IDX_CH * 128, RW), jnp.float32),
            pltpu.SemaphoreType.DMA,
        ],
    )(xp_rows, lags_pad)
    xx = xx.reshape(B, RPP, PAIRS_PAD, RW)

    xr = x[:, 0, :].reshape(B, NF, FS)
    nxt = jnp.concatenate(
        [xr[:, 1:, :OV], jnp.zeros((B, 1, OV), jnp.float32)], axis=1)
    xs = jnp.concatenate([xr, nxt], axis=-1)

    wall = jnp.concatenate([Wk, Wg, Wgg], axis=0).T
    ball = jnp.concatenate(
        [bk, bg, bgg]).astype(jnp.float32).reshape(1, KS + 2)
    win2 = overlap_win.reshape(1, OV)
    win1 = win2[:, ::-1]
    lagsv = lags.reshape(B, NF, 1)

    out = pl.pallas_call(
        _tc_kernel,
        grid=(B,),
        in_specs=[
            pl.BlockSpec((1, RPP, PAIRS_PAD, RW), lambda b: (b, 0, 0, 0)),
            pl.BlockSpec((1, NF, FD), lambda b: (b, 0, 0)),
            pl.BlockSpec((1, NF, 1), lambda b: (b, 0, 0)),
            pl.BlockSpec((1, NF, L_OUT), lambda b: (b, 0, 0)),
            pl.BlockSpec((FD, KS + 2), lambda b: (0, 0)),
            pl.BlockSpec((1, KS + 2), lambda b: (0, 0)),
            pl.BlockSpec((1, OV), lambda b: (0, 0)),
            pl.BlockSpec((1, OV), lambda b: (0, 0)),
        ],
        out_specs=pl.BlockSpec((1, NF, FS), lambda b: (b, 0, 0)),
        out_shape=jax.ShapeDtypeStruct((B, NF, FS), jnp.float32),
    )(xx, features, lagsv, xs, wall, ball, win1, win2)

    return out.reshape(B, 1, NF * FS)


# speedup vs baseline: 7.9834x; 1.0053x over previous
import jax
import jax.numpy as jnp
from jax.experimental import pallas as pl
from jax.experimental.pallas import tpu as pltpu
from jax.experimental.pallas import tpu_sc as plsc

KS = 15
FD = 96
FS = 160
OV = 40
ML = 256
PADL = 7
LGL = 10 * 0.11512925464970229
FGA = 6 * 0.11512925464970229
NF = 100
L_OUT = FS + OV
RW = 128
RPP = 3
XX_W = RPP * RW
XP_LEN = 16512
XP_ROWS = XP_LEN // RW
PAIRS_PAD = 112
NIDX = RPP * PAIRS_PAD
IDX_CH = 3


def _sc_gather(xp_rows, lags_hbm, xx_hbm, lag_v, idx_v, buf_v, sem):
    s = jax.lax.axis_index("s")
    b0 = s * 2
    pltpu.sync_copy(lags_hbm.at[pl.ds(b0, 2)], lag_v)

    zero16 = jnp.zeros((16,), jnp.int32)
    for z in range((2 * IDX_CH * 128 - 2 * NIDX) // 16):
        idx_v[pl.ds(2 * NIDX + z * 16, 16)] = zero16

    for bi in range(2):
        base = (b0 + bi) * XP_ROWS
        for k in range(PAIRS_PAD // 16):
            i16 = jax.lax.iota(jnp.int32, 16) + (k * 16)
            lag16 = lag_v[bi, pl.ds(k * 16, 16)]
            start16 = i16 * FS + ML - lag16
            ar16 = jax.lax.shift_right_logical(start16, 7)
            row16 = jnp.where(i16 < NF, base + ar16, base)
            for r in range(RPP):
                idx_v[pl.ds(bi * NIDX + r * PAIRS_PAD + k * 16, 16)] = (
                    row16 + r)

    cp = pltpu.make_async_copy(xp_rows.at[idx_v], buf_v, sem)
    cp.start()
    cp.wait()

    pltpu.sync_copy(buf_v.at[pl.ds(0, 2 * NIDX)],
                    xx_hbm.at[pl.ds(b0 * NIDX, 2 * NIDX)])


def _tc_kernel(xx_ref, feats_ref, lagsv_ref, xs_ref, wall_ref,
               ball_ref, win1_ref, win2_ref, out_ref):
    feats = feats_ref[0]
    p = jnp.dot(feats, wall_ref[...], preferred_element_type=jnp.float32)
    p = p + ball_ref[...]
    ck = p[:, :KS]
    nrm = jnp.sqrt(jnp.sum(ck * ck, axis=-1, keepdims=True))
    ck = ck / (1e-6 + nrm)
    cg = jnp.exp(-jax.nn.relu(p[:, KS:KS + 1]) + LGL)
    gg = jnp.exp(FGA * jnp.tanh(p[:, KS + 1:KS + 2]))

    w = jnp.concatenate(
        [xx_ref[0, r] for r in range(RPP)], axis=1)[:NF]

    lags = lagsv_ref[0, :, :]
    i_col = jax.lax.broadcasted_iota(jnp.int32, (NF, 1), 0)
    rem = (i_col * FS + ML - lags) & (RW - 1)
    for kbit in (64, 32, 16, 8, 4, 2, 1):
        wsh = jnp.concatenate(
            [w[:, kbit:], jnp.zeros((NF, kbit), jnp.float32)], axis=1)
        w = jnp.where((rem & kbit) != 0, wsh, w)

    nc = jnp.zeros((NF, L_OUT), dtype=jnp.float32)
    for m in range(KS):
        nc = nc + ck[:, m:m + 1] * w[:, m:m + L_OUT]

    nc = gg * (nc * cg + xs_ref[0])

    prev_tail = jnp.concatenate(
        [jnp.zeros((1, OV), jnp.float32), nc[:NF - 1, FS:FS + OV]], axis=0)
    head = nc[:, :OV] * win1_ref[...] + prev_tail * win2_ref[...]
    out_ref[0] = jnp.concatenate([head, nc[:, OV:FS]], axis=1)


def kernel(x, features, lags, Wk, bk, Wg, bg, Wgg, bgg, overlap_win):
    B = x.shape[0]
    lags = lags.astype(jnp.int32)

    xp = jnp.zeros((B, XP_LEN), dtype=jnp.float32)
    xp = jax.lax.dynamic_update_slice(xp, x[:, 0, :], (0, ML + PADL))
    xp_rows = xp.reshape(B * XP_ROWS, RW)

    lags_pad = jnp.pad(lags, ((0, 0), (0, PAIRS_PAD - NF)))

    mesh = plsc.VectorSubcoreMesh(
        core_axis_name="c", subcore_axis_name="s", num_cores=1)
    xx = pl.kernel(
        _sc_gather,
        out_type=jax.ShapeDtypeStruct((B * NIDX, RW), jnp.float32),
        mesh=mesh,
        scratch_types=[
            pltpu.VMEM((2, PAIRS_PAD), jnp.int32),
            pltpu.VMEM((2 * IDX_CH * 128,), jnp.int32),
            pltpu.VMEM((2 * IDX_CH * 128, RW), jnp.float32),
            pltpu.SemaphoreType.DMA,
        ],
    )(xp_rows, lags_pad)
    xx = xx.reshape(B, RPP, PAIRS_PAD, RW)

    xr = x[:, 0, :].reshape(B, NF, FS)
    nxt = jnp.concatenate(
        [xr[:, 1:, :OV], jnp.zeros((B, 1, OV), jnp.float32)], axis=1)
    xs = jnp.concatenate([xr, nxt], axis=-1)

    wall = jnp.concatenate([Wk, Wg, Wgg], axis=0).T
    ball = jnp.concatenate(
        [bk, bg, bgg]).astype(jnp.float32).reshape(1, KS + 2)
    win2 = overlap_win.reshape(1, OV)
    win1 = win2[:, ::-1]
    lagsv = lags.reshape(B, NF, 1)

    out = pl.pallas_call(
        _tc_kernel,
        grid=(B,),
        in_specs=[
            pl.BlockSpec((1, RPP, PAIRS_PAD, RW), lambda b: (b, 0, 0, 0)),
            pl.BlockSpec((1, NF, FD), lambda b: (b, 0, 0)),
            pl.BlockSpec((1, NF, 1), lambda b: (b, 0, 0)),
            pl.BlockSpec((1, NF, L_OUT), lambda b: (b, 0, 0)),
            pl.BlockSpec((FD, KS + 2), lambda b: (0, 0)),
            pl.BlockSpec((1, KS + 2), lambda b: (0, 0)),
            pl.BlockSpec((1, OV), lambda b: (0, 0)),
            pl.BlockSpec((1, OV), lambda b: (0, 0)),
        ],
        out_specs=pl.BlockSpec((1, NF, FS), lambda b: (b, 0, 0)),
        out_shape=jax.ShapeDtypeStruct((B, NF, FS), jnp.float32),
    )(xx, features, lagsv, xs, wall, ball, win1, win2)

    return out.reshape(B, 1, NF * FS)


# speedup vs baseline: 9.6117x; 1.2040x over previous
import jax
import jax.numpy as jnp
from jax.experimental import pallas as pl
from jax.experimental.pallas import tpu as pltpu
from jax.experimental.pallas import tpu_sc as plsc

KS = 15
FD = 96
FS = 160
OV = 40
ML = 256
PADL = 7
LGL = 10 * 0.11512925464970229
FGA = 6 * 0.11512925464970229
NF = 100
L_OUT = FS + OV
RW = 128
RPP = 3
XP_LEN = 16512
XP_ROWS = XP_LEN // RW
NPB = 112
NIDX = RPP * NPB
IDXN = 2 * 384


def _sc_gather(xp_rows, lags_hbm, xx0, xx1, xx2, lag_v, idx_v, buf_v, sem):
    s = jax.lax.axis_index("s")
    b0 = s * 2
    pltpu.sync_copy(lags_hbm.at[pl.ds(b0, 2)], lag_v)

    zero16 = jnp.zeros((16,), jnp.int32)
    for z in range((IDXN - 2 * NIDX) // 16):
        idx_v[pl.ds(2 * NIDX + z * 16, 16)] = zero16

    for bi in range(2):
        base = (b0 + bi) * XP_ROWS
        for k in range(NPB // 16):
            i16 = jax.lax.iota(jnp.int32, 16) + (k * 16)
            lag16 = lag_v[bi, pl.ds(k * 16, 16)]
            start16 = i16 * FS + ML - lag16
            ar16 = jax.lax.shift_right_logical(start16, 7)
            row16 = jnp.where(i16 < NF, base + ar16, base)
            for r in range(RPP):
                idx_v[pl.ds(bi * NIDX + r * NPB + k * 16, 16)] = row16 + r

    cp = pltpu.make_async_copy(xp_rows.at[idx_v], buf_v, sem)
    cp.start()
    cp.wait()

    outs = (xx0, xx1, xx2)
    for bi in range(2):
        for r in range(RPP):
            pltpu.sync_copy(
                buf_v.at[pl.ds(bi * NIDX + r * NPB, NPB)],
                outs[r].at[pl.ds((b0 + bi) * NPB, NPB)])


def _tc_kernel(xx0_ref, xx1_ref, xx2_ref, feats_ref, lagsv_ref, xs_ref,
               wall_ref, ball_ref, win1_ref, win2_ref, out_ref):
    R = xx0_ref.shape[0]

    p = jnp.dot(feats_ref[...], wall_ref[...],
                preferred_element_type=jnp.float32)
    p = p + ball_ref[...]
    ck = p[:, :KS]
    nrm = jnp.sqrt(jnp.sum(ck * ck, axis=-1, keepdims=True))
    ck = ck / (1e-6 + nrm)
    cg = jnp.exp(-jax.nn.relu(p[:, KS:KS + 1]) + LGL)
    gg = jnp.exp(FGA * jnp.tanh(p[:, KS + 1:KS + 2]))

    w = jnp.concatenate(
        [xx0_ref[...], xx1_ref[...], xx2_ref[...]], axis=1)
    lags = lagsv_ref[...]
    row_iota = jax.lax.broadcasted_iota(jnp.int32, (R, 1), 0)
    i_col = jax.lax.rem(row_iota, NPB)
    rem = (i_col * FS + ML - lags) & (RW - 1)
    for kbit in (64, 32, 16, 8, 4, 2, 1):
        wsh = jnp.concatenate(
            [w[:, kbit:], jnp.zeros((R, kbit), jnp.float32)], axis=1)
        w = jnp.where((rem & kbit) != 0, wsh, w)

    nc = jnp.zeros((R, L_OUT), dtype=jnp.float32)
    for m in range(KS):
        nc = nc + ck[:, m:m + 1] * w[:, m:m + L_OUT]

    nc = gg * (nc * cg + xs_ref[...])

    prev_tail = jnp.concatenate(
        [jnp.zeros((1, OV), jnp.float32), nc[:R - 1, FS:FS + OV]], axis=0)
    prev_tail = jnp.where(i_col == 0, 0.0, prev_tail)
    head = nc[:, :OV] * win1_ref[...] + prev_tail * win2_ref[...]
    out_ref[...] = jnp.concatenate([head, nc[:, OV:FS]], axis=1)


def kernel(x, features, lags, Wk, bk, Wg, bg, Wgg, bgg, overlap_win):
    B = x.shape[0]
    R = B * NPB
    lags = lags.astype(jnp.int32)

    xp = jnp.zeros((B, XP_LEN), dtype=jnp.float32)
    xp = jax.lax.dynamic_update_slice(xp, x[:, 0, :], (0, ML + PADL))
    xp_rows = xp.reshape(B * XP_ROWS, RW)

    lags_pad = jnp.pad(lags, ((0, 0), (0, NPB - NF)))

    mesh = plsc.VectorSubcoreMesh(
        core_axis_name="c", subcore_axis_name="s", num_cores=1)
    xx0, xx1, xx2 = pl.kernel(
        _sc_gather,
        out_type=[jax.ShapeDtypeStruct((R, RW), jnp.float32)] * RPP,
        mesh=mesh,
        scratch_types=[
            pltpu.VMEM((2, NPB), jnp.int32),
            pltpu.VMEM((IDXN,), jnp.int32),
            pltpu.VMEM((IDXN, RW), jnp.float32),
            pltpu.SemaphoreType.DMA,
        ],
    )(xp_rows, lags_pad)

    xr = x[:, 0, :].reshape(B, NF, FS)
    nxt = jnp.concatenate(
        [xr[:, 1:, :OV], jnp.zeros((B, 1, OV), jnp.float32)], axis=1)
    xs = jnp.concatenate([xr, nxt], axis=-1)
    xs_pad = jnp.pad(xs, ((0, 0), (0, NPB - NF), (0, 0))).reshape(R, L_OUT)
    feats_pad = jnp.pad(
        features, ((0, 0), (0, NPB - NF), (0, 0))).reshape(R, FD)
    lagsv = jnp.pad(lags, ((0, 0), (0, NPB - NF))).reshape(R, 1)

    wall = jnp.concatenate([Wk, Wg, Wgg], axis=0).T
    ball = jnp.concatenate(
        [bk, bg, bgg]).astype(jnp.float32).reshape(1, KS + 2)
    win2 = overlap_win.reshape(1, OV)
    win1 = win2[:, ::-1]

    RC = 8 * NPB
    out = pl.pallas_call(
        _tc_kernel,
        grid=(R // RC,),
        in_specs=[
            pl.BlockSpec((RC, RW), lambda i: (i, 0)),
            pl.BlockSpec((RC, RW), lambda i: (i, 0)),
            pl.BlockSpec((RC, RW), lambda i: (i, 0)),
            pl.BlockSpec((RC, FD), lambda i: (i, 0)),
            pl.BlockSpec((RC, 1), lambda i: (i, 0)),
            pl.BlockSpec((RC, L_OUT), lambda i: (i, 0)),
            pl.BlockSpec((FD, KS + 2), lambda i: (0, 0)),
            pl.BlockSpec((1, KS + 2), lambda i: (0, 0)),
            pl.BlockSpec((1, OV), lambda i: (0, 0)),
            pl.BlockSpec((1, OV), lambda i: (0, 0)),
        ],
        out_specs=pl.BlockSpec((RC, FS), lambda i: (i, 0)),
        out_shape=jax.ShapeDtypeStruct((R, FS), jnp.float32),
    )(xx0, xx1, xx2, feats_pad, lagsv, xs_pad, wall, ball, win1, win2)

    out = out.reshape(B, NPB, FS)[:, :NF, :]
    return out.reshape(B, 1, NF * FS)
